# Initial kernel scaffold; baseline (speedup 1.0000x reference)
import jax
import jax.numpy as jnp
from jax.experimental import pallas as pl


def kernel(x, target, anchors):
    raise NotImplementedError("write your pallas kernel here")


# speedup vs baseline: 3.7385x; 3.7385x over previous
import functools

import jax
import jax.numpy as jnp
from jax import lax
from jax.experimental import pallas as pl
from jax.experimental.pallas import tpu as pltpu

N_CLASS = 80
N_ANCHOR = 3
G = 76
GG = G * G
B = 16
CH = N_CLASS + 5
STRIDE = 8.0
THRESH = 0.5
NO_OBJ_W = 100.0
NCELL = float(B * N_ANCHOR * GG)


def _tc_body(x_ref, anch_ref, out_ref, sum_ref):
    b = pl.program_id(0)
    a = pl.program_id(1)
    X = x_ref[0, 0]
    S = jax.nn.sigmoid(X)
    col = lax.broadcasted_iota(jnp.int32, (1, GG), 1)
    gx = (col % G).astype(jnp.float32)
    gy = (col // G).astype(jnp.float32)
    aw = anch_ref[a, 0]
    ah = anch_ref[a, 1]
    r0 = (S[0:1] + gx) * STRIDE
    r1 = (S[1:2] + gy) * STRIDE
    r2 = jnp.exp(S[2:3]) * (aw * STRIDE)
    r3 = jnp.exp(S[3:4]) * (ah * STRIDE)
    top = jnp.concatenate([r0, r1, r2, r3, S[4:]], axis=0)
    out_ref[0, 0] = top.T
    z = X[4:5]
    sp = jnp.maximum(z, 0.0) + jnp.log1p(jnp.exp(-jnp.abs(z)))
    part = jnp.sum(jnp.minimum(sp, 100.0))

    @pl.when((b == 0) & (a == 0))
    def _():
        sum_ref[0, 0] = 0.0

    sum_ref[0, 0] += part


def _tc_transform(x4, anchors, interpret=False):
    return pl.pallas_call(
        _tc_body,
        grid=(B, N_ANCHOR),
        in_specs=[
            pl.BlockSpec((1, 1, CH, GG), lambda b, a: (b, a, 0, 0)),
            pl.BlockSpec(memory_space=pltpu.SMEM),
        ],
        out_specs=[
            pl.BlockSpec((1, 1, GG, CH), lambda b, a: (b, a, 0, 0)),
            pl.BlockSpec(memory_space=pltpu.SMEM),
        ],
        out_shape=[
            jax.ShapeDtypeStruct((B, N_ANCHOR, GG, CH), jnp.float32),
            jax.ShapeDtypeStruct((1, 1), jnp.float32),
        ],
        interpret=interpret,
    )(x4, anchors)


def _poly_log1p(w):
    s = w / (2.0 + w)
    s2 = s * s
    return 2.0 * s * (1.0 + s2 * (1.0 / 3.0 + s2 * (0.2 + s2 * (1.0 / 7.0 + s2 / 9.0))))


def _poly_log(x):
    bits = lax.bitcast_convert_type(x, jnp.int32)
    e = ((bits >> 23) & 0xFF) - 127
    mbits = (bits & 0x7FFFFF) | (127 << 23)
    m = lax.bitcast_convert_type(mbits, jnp.float32)
    big = m > 1.4142135
    m = jnp.where(big, m * 0.5, m)
    e = e + big.astype(jnp.int32)
    s = (m - 1.0) / (m + 1.0)
    s2 = s * s
    lm = 2.0 * s * (1.0 + s2 * (1.0 / 3.0 + s2 * (0.2 + s2 * (1.0 / 7.0 + s2 / 9.0))))
    return e.astype(jnp.float32) * 0.6931471805599453 + lm


def _softplus_c(t):
    w = jnp.exp(-jnp.abs(t))
    return jnp.minimum(jnp.maximum(t, 0.0) + _poly_log1p(w), 100.0)


def _sigmoid_c(z):
    return 1.0 / (1.0 + jnp.exp(-z))


def _loss_parts_jnp(x, target, anchors):
    t0 = target[:, 0]; t1 = target[:, 1]; tw = target[:, 2]; th = target[:, 3]
    tcx = t0 * G; tcy = t1 * G
    tci = tcx.astype(jnp.int32); tcj = tcy.astype(jnp.int32)
    fx = tcx - tci.astype(jnp.float32); fy = tcy - tcj.astype(jnp.float32)
    colc = tci * G + tcj
    ious = []
    for a in range(N_ANCHOR):
        inter = jnp.minimum(anchors[a, 0], tw) * jnp.minimum(anchors[a, 1], th)
        union = tw * th + anchors[a, 0] * anchors[a, 1] - inter
        ious.append(inter / union)
    best = jnp.where(ious[1] > ious[0], 1, 0)
    best = jnp.where(ious[2] > jnp.maximum(ious[0], ious[1]), 2, best)
    aw_b = jnp.where(best == 0, anchors[0, 0], jnp.where(best == 1, anchors[1, 0], anchors[2, 0]))
    ah_b = jnp.where(best == 0, anchors[0, 1], jnp.where(best == 1, anchors[1, 1], anchors[2, 1]))
    bi = jnp.arange(B)
    xf = x.reshape(B, N_ANCHOR * CH, GG)

    def val(j):
        return xf[bi, best * CH + j, colc]

    d0 = _sigmoid_c(val(0)) - fx
    d1 = _sigmoid_c(val(1)) - fy
    d2 = _sigmoid_c(val(2)) - _poly_log(tw / aw_b + 1e-16)
    d3 = _sigmoid_c(val(3)) - _poly_log(th / ah_b + 1e-16)
    box_sum = jnp.sum(d0 * d0 + d1 * d1 + d2 * d2 + d3 * d3)
    objconf_sum = jnp.sum(_softplus_c(-val(4)))
    acc = jnp.zeros((B,), jnp.float32)
    for j in range(5, CH):
        v = val(j)
        t = target[:, j - 1]
        acc = acc + t * _softplus_c(-v) + (1.0 - t) * _softplus_c(v)
    class_sum = jnp.sum(acc)
    corr = jnp.zeros((B,), jnp.float32)
    cnt = jnp.zeros((B,), jnp.float32)
    for a in range(N_ANCHOR):
        zc = xf[bi, a * CH + 4, colc]
        rem = (ious[a] > THRESH) | (best == a)
        corr = corr + jnp.where(rem, _softplus_c(zc), 0.0)
        cnt = cnt + rem.astype(jnp.float32)
    return box_sum, objconf_sum, class_sum, jnp.sum(corr), jnp.sum(cnt)


def kernel(x, target, anchors):
    x4 = x.reshape(B, N_ANCHOR, CH, GG)
    out4, s_total = _tc_transform(x4, anchors)
    output = out4.reshape(B, N_ANCHOR * GG, CH)
    box_sum, objconf_sum, class_sum, corr_sum, removed = _loss_parts_jnp(x, target, anchors)
    loss = (box_sum / B + objconf_sum / B
            + NO_OBJ_W * (s_total[0, 0] - corr_sum) / (NCELL - removed)
            + class_sum / (B * N_CLASS))
    return output, loss
